# Initial kernel scaffold; baseline (speedup 1.0000x reference)
import jax
import jax.numpy as jnp
from jax.experimental import pallas as pl


def kernel(x, edge_index, edge_attr, batch, W_pre, b_pre, W1, b1, bn1_g, bn1_b, W2, b2, We, be, bn_g, bn_b, eps_gin, W_out, b_out):
    raise NotImplementedError("write your pallas kernel here")


# speedup vs baseline: 2.6520x; 2.6520x over previous
import functools

import jax
import jax.numpy as jnp
from jax import lax
from jax.experimental import pallas as pl
from jax.experimental.pallas import tpu as pltpu
from jax.experimental.pallas import tpu_sc as plsc

N = 10000
E = 160000
D = 512
L = 5
G = 128
NL = 68
DC = 128
NSL = 4
NS = 16
EPT = E // NS
K = 50
CH = EPT // K
RPT = N // NS
WB = 125
NSLOT = 2
MSLOT = 4
W = 64


def _sc_agg_body(hp2, ep, src8, dst3, out, srcv, dstv, ebufs, gbufs, mbufs,
                 wb16, wb32, acc, esems, gsems, ssems):
    ci = lax.axis_index("c")
    s = lax.axis_index("s")
    row0 = s * RPT
    ebase = s * EPT

    zb32 = jnp.zeros((32,), jnp.bfloat16)

    def _zero_rows_bf16(buf, nrows):
        def zrow(r, _):
            for u in range(4):
                buf[r, pl.ds(u * 32, 32)] = zb32
            return 0
        lax.fori_loop(0, nrows, zrow, 0, unroll=4)

    def issue_eg(c2, qq, j, k):
        pltpu.async_copy(ep.at[c2, pl.ds(ebase + j * K, K),
                               pl.ds(qq * W, W)], ebufs[k], esems[k])
        pltpu.async_copy(hp2.at[srcv.at[j]], gbufs[k], gsems[k])

    def compute(k, m):
        eb, gb, mb = ebufs[k], gbufs[k], mbufs[m]

        def row(r, _):
            for u in range(4):
                sl = pl.ds(u * 16, 16)
                ev = plsc.bitcast(eb[r, sl], jnp.bfloat16)
                gv = plsc.bitcast(gb[r, sl], jnp.bfloat16)
                mb[r, pl.ds(u * 32, 32)] = jnp.maximum(ev + gv, 0.0)
            return 0
        lax.fori_loop(0, K, row, 0, unroll=8)

    def wait_eg(k):
        pltpu.make_async_copy(ep.at[0, pl.ds(0, K), pl.ds(0, W)], ebufs[k],
                              esems[k]).wait()
        pltpu.make_async_copy(hp2.at[srcv.at[0]], gbufs[k], gsems[k]).wait()

    def issue_scatter(j, m):
        pltpu.async_copy(mbufs[m], acc.at[dstv.at[j]], ssems[m], add=True)

    def wait_scatter(m):
        pltpu.make_async_copy(mbufs[m], acc.at[dstv.at[0]], ssems[m]).wait()

    for tl in range(2):
        c2 = ci
        qq = tl
        g = c2 * 2 + qq

        for m in range(MSLOT):
            _zero_rows_bf16(mbufs[m], K)
        _zero_rows_bf16(wb16, WB)
        for r in range(RPT // WB):
            pltpu.sync_copy(wb16, acc.at[pl.ds(row0 + r * WB, WB)])

        pltpu.sync_copy(src8.at[c2, qq, s], srcv)
        if tl == 0:
            pltpu.sync_copy(dst3.at[s], dstv)

        plsc.subcore_barrier()

        for m in range(MSLOT):
            issue_scatter(0, m)
        for k in range(NSLOT):
            issue_eg(c2, qq, k, k)

        def chunk_group(u, _):
            for t in range(MSLOT):
                j = MSLOT * u + t
                k = t % NSLOT
                m = t
                wait_eg(k)
                wait_scatter(m)
                compute(k, m)
                issue_scatter(j, m)
                issue_eg(c2, qq, j + NSLOT, k)
            return 0
        lax.fori_loop(0, CH // MSLOT - 1, chunk_group, 0)

        for t in range(MSLOT):
            j = CH - MSLOT + t
            k = t % NSLOT
            m = t
            wait_eg(k)
            wait_scatter(m)
            compute(k, m)
            issue_scatter(j, m)
            if j + NSLOT < CH:
                issue_eg(c2, qq, j + NSLOT, k)
        for m in range(MSLOT):
            wait_scatter(m)

        plsc.subcore_barrier()

        for r in range(RPT // WB):
            rows = pl.ds(row0 + r * WB, WB)
            pltpu.sync_copy(acc.at[rows], wb16)

            def unpack_row(rr, _):
                for u in range(4):
                    a, b = plsc.unpack(wb16[rr, pl.ds(u * 32, 32)],
                                       format=plsc.PackFormat.INTERLEAVED)
                    wb32[rr, pl.ds(u * 16, 16)] = a
                    wb32[rr, pl.ds(64 + u * 16, 16)] = b
                return 0
            lax.fori_loop(0, WB, unpack_row, 0, unroll=4)
            pltpu.sync_copy(wb32, out.at[g, rows])

        plsc.subcore_barrier()


@jax.jit
def _sc_agg(hp2, ep, src8, dst3):
    def body(hp2r, epr, src8r, dst3r, outr, srcvr, dstvr, *rest):
        ebufs = rest[0:NSLOT]
        gbufs = rest[NSLOT:2 * NSLOT]
        mbufs = rest[2 * NSLOT:2 * NSLOT + MSLOT]
        wb16r = rest[2 * NSLOT + MSLOT]
        wb32r = rest[2 * NSLOT + MSLOT + 1]
        accr = rest[2 * NSLOT + MSLOT + 2]
        sems = rest[2 * NSLOT + MSLOT + 3:]
        esems = sems[0:NSLOT]
        gsems = sems[NSLOT:2 * NSLOT]
        ssems = sems[2 * NSLOT:]
        _sc_agg_body(hp2r, epr, src8r, dst3r, outr, srcvr, dstvr, ebufs,
                     gbufs, mbufs, wb16r, wb32r, accr, esems, gsems, ssems)

    return pl.kernel(
        body,
        out_type=jax.ShapeDtypeStruct((NSL, N, DC), jnp.float32),
        mesh=plsc.VectorSubcoreMesh(core_axis_name="c", subcore_axis_name="s"),
        compiler_params=pltpu.CompilerParams(use_tc_tiling_on_sc=False,
                                             needs_layout_passes=False),
        scratch_types=(
            [pltpu.VMEM((CH, K), jnp.int32),
             pltpu.VMEM((CH, K), jnp.int32)]
            + [pltpu.VMEM((K, W), jnp.int32)] * (2 * NSLOT)
            + [pltpu.VMEM((K, 2 * W), jnp.bfloat16)] * MSLOT
            + [pltpu.VMEM((WB, 2 * W), jnp.bfloat16),
               pltpu.VMEM((WB, DC), jnp.float32),
               pltpu.VMEM_SHARED((N, 2 * W), jnp.bfloat16)]
            + [pltpu.SemaphoreType.DMA] * (2 * NSLOT + MSLOT)
        ),
    )(hp2, ep, src8, dst3)


def _pack_pair(lo, hi):
    lo16 = jax.lax.bitcast_convert_type(lo.astype(jnp.bfloat16),
                                        jnp.uint16).astype(jnp.uint32)
    hi16 = jax.lax.bitcast_convert_type(hi.astype(jnp.bfloat16),
                                        jnp.uint16).astype(jnp.uint32)
    return ((hi16 << 16) | lo16).astype(jnp.int32)


def _pack_256(o):
    return jnp.concatenate(
        [_pack_pair(o[:, 0:64], o[:, 64:128]),
         _pack_pair(o[:, 128:192], o[:, 192:256])], axis=1)


def _pre_body(x_ref, w_ref, b_ref, o_ref, hp_ref):
    o = (jnp.dot(x_ref[...], w_ref[0],
                 preferred_element_type=jnp.float32) + b_ref[0])
    o_ref[0] = o[:, 0:128]
    o_ref[1] = o[:, 128:256]
    hp_ref[0] = _pack_256(o)


@jax.jit
def _pre(x, w2, b2):
    return pl.pallas_call(
        _pre_body,
        grid=(2,),
        in_specs=[
            pl.BlockSpec((N, 128), lambda c: (0, 0)),
            pl.BlockSpec((1, 128, 2 * DC), lambda c: (c, 0, 0)),
            pl.BlockSpec((1, 1, 2 * DC), lambda c: (c, 0, 0)),
        ],
        out_specs=[
            pl.BlockSpec((2, N, DC), lambda c: (c, 0, 0)),
            pl.BlockSpec((1, N, DC), lambda c: (c, 0, 0)),
        ],
        out_shape=[
            jax.ShapeDtypeStruct((NSL, N, DC), jnp.float32),
            jax.ShapeDtypeStruct((2, N, DC), jnp.int32),
        ],
    )(x, w2, b2)


_BE = 8000


def _edge_body(ea_ref, w_ref, b_ref, o_ref):
    o = (jnp.dot(ea_ref[...], w_ref[0],
                 preferred_element_type=jnp.float32) + b_ref[0])
    o_ref[0] = _pack_256(o)


@jax.jit
def _edge(ea, w2, b2):
    return pl.pallas_call(
        _edge_body,
        grid=(2, E // _BE),
        in_specs=[
            pl.BlockSpec((_BE, 16), lambda c, j: (j, 0)),
            pl.BlockSpec((1, 16, 2 * DC), lambda c, j: (c, 0, 0)),
            pl.BlockSpec((1, 1, 2 * DC), lambda c, j: (c, 0, 0)),
        ],
        out_specs=pl.BlockSpec((1, _BE, DC), lambda c, j: (c, j, 0)),
        out_shape=jax.ShapeDtypeStruct((2, E, DC), jnp.int32),
    )(ea, w2, b2)


_BN = 1000


def _mlp_body(eps_ref, h_ref, a_ref, w1_ref, b1_ref, w2_ref, b2_ref, o_ref,
              hp_ref):
    ep = eps_ref[0, 0]
    relu_flag = eps_ref[0, 1]
    z1 = jnp.dot(ep * h_ref[0] + a_ref[0], w1_ref[0],
                 preferred_element_type=jnp.float32)
    for c in range(1, NSL):
        z1 += jnp.dot(ep * h_ref[c] + a_ref[c], w1_ref[c],
                      preferred_element_type=jnp.float32)
    z1 = jnp.maximum(z1 + b1_ref[...], 0.0)
    os = []
    for c in range(NSL):
        o = (jnp.dot(z1, w2_ref[c], preferred_element_type=jnp.float32)
             + b2_ref[c:c + 1, :])
        o = jnp.where(relu_flag > 0.5, jnp.maximum(o, 0.0), o)
        o_ref[c] = o
        os.append(o)
    for c2 in range(2):
        hp_ref[c2] = _pack_256(jnp.concatenate(
            [os[2 * c2], os[2 * c2 + 1]], axis=1))


@jax.jit
def _mlp(scal, h4, agg4, w1r, b1f, w2r, b2f):
    return pl.pallas_call(
        _mlp_body,
        grid=(N // _BN,),
        in_specs=[
            pl.BlockSpec(memory_space=pltpu.SMEM),
            pl.BlockSpec((NSL, _BN, DC), lambda i: (0, i, 0)),
            pl.BlockSpec((NSL, _BN, DC), lambda i: (0, i, 0)),
            pl.BlockSpec((NSL, DC, 2 * D), lambda i: (0, 0, 0)),
            pl.BlockSpec((1, 2 * D), lambda i: (0, 0)),
            pl.BlockSpec((NSL, 2 * D, DC), lambda i: (0, 0, 0)),
            pl.BlockSpec((NSL, DC), lambda i: (0, 0)),
        ],
        out_specs=[
            pl.BlockSpec((NSL, _BN, DC), lambda i: (0, i, 0)),
            pl.BlockSpec((2, _BN, DC), lambda i: (0, i, 0)),
        ],
        out_shape=[
            jax.ShapeDtypeStruct((NSL, N, DC), jnp.float32),
            jax.ShapeDtypeStruct((2, N, DC), jnp.int32),
        ],
    )(scal, h4, agg4, w1r, b1f, w2r, b2f)


def _pool_body(b_ref, h_ref, wo_ref, bo_ref, o_ref, sums, cnt):
    i = pl.program_id(0)

    @pl.when(i == 0)
    def _init():
        sums[...] = jnp.zeros_like(sums)
        cnt[...] = jnp.zeros_like(cnt)

    pt = (lax.broadcasted_iota(jnp.int32, (G, _BN), 0)
          == b_ref[0]).astype(jnp.float32)
    cnt[...] += jnp.dot(pt, jnp.ones((_BN, 128), jnp.float32),
                        preferred_element_type=jnp.float32)
    for c in range(NSL):
        sums[c] += jnp.dot(pt, h_ref[c], preferred_element_type=jnp.float32)

    @pl.when(i == N // _BN - 1)
    def _fin():
        inv = 1.0 / jnp.maximum(cnt[...], 1.0)
        o = bo_ref[...] + jnp.dot(sums[0] * inv, wo_ref[0],
                                  preferred_element_type=jnp.float32)
        for c in range(1, NSL):
            o += jnp.dot(sums[c] * inv, wo_ref[c],
                         preferred_element_type=jnp.float32)
        o_ref[...] = o


@jax.jit
def _pool(batch2, h4, wo4, bo2):
    return pl.pallas_call(
        _pool_body,
        grid=(N // _BN,),
        in_specs=[
            pl.BlockSpec((1, 1, _BN), lambda i: (i, 0, 0)),
            pl.BlockSpec((NSL, _BN, DC), lambda i: (0, i, 0)),
            pl.BlockSpec((NSL, DC, NL), lambda i: (0, 0, 0)),
            pl.BlockSpec((1, NL), lambda i: (0, 0)),
        ],
        out_specs=pl.BlockSpec((G, NL), lambda i: (0, 0)),
        out_shape=jax.ShapeDtypeStruct((G, NL), jnp.float32),
        scratch_shapes=[
            pltpu.VMEM((NSL, G, DC), jnp.float32),
            pltpu.VMEM((G, DC), jnp.float32),
        ],
    )(batch2, h4, wo4, bo2)


def kernel(x, edge_index, edge_attr, batch, W_pre, b_pre, W1, b1, bn1_g,
           bn1_b, W2, b2, We, be, bn_g, bn_b, eps_gin, W_out, b_out):
    f32 = jnp.float32
    src = edge_index[0]
    dst = edge_index[1]
    src8 = (2 * src.reshape(1, 1, NS, CH, K)
            + jnp.arange(2, dtype=jnp.int32).reshape(1, 2, 1, 1, 1)
            + (jnp.arange(2, dtype=jnp.int32) * 2 * N).reshape(2, 1, 1, 1, 1))
    dst3 = dst.reshape(NS, CH, K)
    batch2 = batch.reshape(N // _BN, 1, _BN)

    wpre2 = W_pre.reshape(128, 2, 2 * DC).transpose(1, 0, 2)
    bpre2 = b_pre.reshape(2, 1, 2 * DC)
    w1f = W1 * bn1_g[:, None, :]
    b1f = b1 * bn1_g + bn1_b
    w2f = W2 * bn_g[:, None, :]
    b2f = b2 * bn_g + bn_b
    w1r = w1f.reshape(L, NSL, DC, 2 * D)
    w2r = w2f.reshape(L, 2 * D, NSL, DC).transpose(0, 2, 1, 3)
    we2 = We.reshape(L, 16, 2, 2 * DC).transpose(0, 2, 1, 3)
    be2 = be.reshape(L, 2, 1, 2 * DC)
    wo4 = W_out.reshape(NSL, DC, NL)
    bo2 = b_out.reshape(1, NL)
    b1f2 = b1f.reshape(L, 1, 2 * D)
    b2f2 = b2f.reshape(L, NSL, DC)
    relu_flags = (jnp.arange(L) < L - 1).astype(f32)
    scal = jnp.stack([1.0 + eps_gin.astype(f32), relu_flags], axis=1)

    h4, hp = _pre(x, wpre2, bpre2)

    def layer(l, carry):
        h4, hp = carry
        idx = functools.partial(lax.dynamic_index_in_dim, index=l, axis=0,
                                keepdims=False)
        ep = _edge(edge_attr, idx(we2), idx(be2))
        agg4 = _sc_agg(hp.reshape(2 * N * 2, W), ep, src8, dst3)
        h4n, hpn = _mlp(idx(scal).reshape(1, 2), h4, agg4, idx(w1r),
                        idx(b1f2), idx(w2r), idx(b2f2))
        return (h4n, hpn)

    h4, hp = lax.fori_loop(0, L, layer, (h4, hp))
    return _pool(batch2, h4, wo4, bo2)


# speedup vs baseline: 2.6999x; 1.0181x over previous
import functools

import jax
import jax.numpy as jnp
from jax import lax
from jax.experimental import pallas as pl
from jax.experimental.pallas import tpu as pltpu
from jax.experimental.pallas import tpu_sc as plsc

N = 10000
E = 160000
D = 512
L = 5
G = 128
NL = 68
DC = 128
NSL = 4
NS = 16
EPT = E // NS
K = 100
CH = EPT // K
RPT = N // NS
WB = 125
NSLOT = 2
MSLOT = 2
W = 64


def _sc_agg_body(hp2, ep, src8, dst3, out, srcv, dstv, ebufs, gbufs, mbufs,
                 wb16, wb32, acc, esems, gsems, ssems):
    ci = lax.axis_index("c")
    s = lax.axis_index("s")
    row0 = s * RPT
    ebase = s * EPT

    zb32 = jnp.zeros((32,), jnp.bfloat16)

    def _zero_rows_bf16(buf, nrows):
        def zrow(r, _):
            for u in range(4):
                buf[r, pl.ds(u * 32, 32)] = zb32
            return 0
        lax.fori_loop(0, nrows, zrow, 0, unroll=4)

    def issue_eg(c2, qq, j, k):
        pltpu.async_copy(ep.at[c2, pl.ds(ebase + j * K, K),
                               pl.ds(qq * W, W)], ebufs[k], esems[k])
        pltpu.async_copy(hp2.at[srcv.at[j]], gbufs[k], gsems[k])

    def compute(k, m):
        eb, gb, mb = ebufs[k], gbufs[k], mbufs[m]

        def row(r, _):
            for u in range(4):
                sl = pl.ds(u * 16, 16)
                ev = plsc.bitcast(eb[r, sl], jnp.bfloat16)
                gv = plsc.bitcast(gb[r, sl], jnp.bfloat16)
                mb[r, pl.ds(u * 32, 32)] = jnp.maximum(ev + gv, 0.0)
            return 0
        lax.fori_loop(0, K, row, 0, unroll=8)

    def wait_eg(k):
        pltpu.make_async_copy(ep.at[0, pl.ds(0, K), pl.ds(0, W)], ebufs[k],
                              esems[k]).wait()
        pltpu.make_async_copy(hp2.at[srcv.at[0]], gbufs[k], gsems[k]).wait()

    def issue_scatter(j, m):
        pltpu.async_copy(mbufs[m], acc.at[dstv.at[j]], ssems[m], add=True)

    def wait_scatter(m):
        pltpu.make_async_copy(mbufs[m], acc.at[dstv.at[0]], ssems[m]).wait()

    for tl in range(2):
        c2 = ci
        qq = tl
        g = c2 * 2 + qq

        for m in range(MSLOT):
            _zero_rows_bf16(mbufs[m], K)
        _zero_rows_bf16(wb16, WB)
        for r in range(RPT // WB):
            pltpu.sync_copy(wb16, acc.at[pl.ds(row0 + r * WB, WB)])

        pltpu.sync_copy(src8.at[c2, qq, s], srcv)
        if tl == 0:
            pltpu.sync_copy(dst3.at[s], dstv)

        plsc.subcore_barrier()

        for m in range(MSLOT):
            issue_scatter(0, m)
        for k in range(NSLOT):
            issue_eg(c2, qq, k, k)

        def chunk_group(u, _):
            for t in range(MSLOT):
                j = MSLOT * u + t
                k = t % NSLOT
                m = t
                wait_eg(k)
                wait_scatter(m)
                compute(k, m)
                issue_scatter(j, m)
                issue_eg(c2, qq, j + NSLOT, k)
            return 0
        lax.fori_loop(0, CH // MSLOT - 1, chunk_group, 0)

        for t in range(MSLOT):
            j = CH - MSLOT + t
            k = t % NSLOT
            m = t
            wait_eg(k)
            wait_scatter(m)
            compute(k, m)
            issue_scatter(j, m)
            if j + NSLOT < CH:
                issue_eg(c2, qq, j + NSLOT, k)
        for m in range(MSLOT):
            wait_scatter(m)

        plsc.subcore_barrier()

        for r in range(RPT // WB):
            rows = pl.ds(row0 + r * WB, WB)
            pltpu.sync_copy(acc.at[rows], wb16)

            def unpack_row(rr, _):
                for u in range(4):
                    a, b = plsc.unpack(wb16[rr, pl.ds(u * 32, 32)],
                                       format=plsc.PackFormat.INTERLEAVED)
                    wb32[rr, pl.ds(u * 16, 16)] = a
                    wb32[rr, pl.ds(64 + u * 16, 16)] = b
                return 0
            lax.fori_loop(0, WB, unpack_row, 0, unroll=4)
            pltpu.sync_copy(wb32, out.at[g, rows])

        plsc.subcore_barrier()


@jax.jit
def _sc_agg(hp2, ep, src8, dst3):
    def body(hp2r, epr, src8r, dst3r, outr, srcvr, dstvr, *rest):
        ebufs = rest[0:NSLOT]
        gbufs = rest[NSLOT:2 * NSLOT]
        mbufs = rest[2 * NSLOT:2 * NSLOT + MSLOT]
        wb16r = rest[2 * NSLOT + MSLOT]
        wb32r = rest[2 * NSLOT + MSLOT + 1]
        accr = rest[2 * NSLOT + MSLOT + 2]
        sems = rest[2 * NSLOT + MSLOT + 3:]
        esems = sems[0:NSLOT]
        gsems = sems[NSLOT:2 * NSLOT]
        ssems = sems[2 * NSLOT:]
        _sc_agg_body(hp2r, epr, src8r, dst3r, outr, srcvr, dstvr, ebufs,
                     gbufs, mbufs, wb16r, wb32r, accr, esems, gsems, ssems)

    return pl.kernel(
        body,
        out_type=jax.ShapeDtypeStruct((NSL, N, DC), jnp.float32),
        mesh=plsc.VectorSubcoreMesh(core_axis_name="c", subcore_axis_name="s"),
        compiler_params=pltpu.CompilerParams(use_tc_tiling_on_sc=False,
                                             needs_layout_passes=False),
        scratch_types=(
            [pltpu.VMEM((CH, K), jnp.int32),
             pltpu.VMEM((CH, K), jnp.int32)]
            + [pltpu.VMEM((K, W), jnp.int32)] * (2 * NSLOT)
            + [pltpu.VMEM((K, 2 * W), jnp.bfloat16)] * MSLOT
            + [pltpu.VMEM((WB, 2 * W), jnp.bfloat16),
               pltpu.VMEM((WB, DC), jnp.float32),
               pltpu.VMEM_SHARED((N, 2 * W), jnp.bfloat16)]
            + [pltpu.SemaphoreType.DMA] * (2 * NSLOT + MSLOT)
        ),
    )(hp2, ep, src8, dst3)


def _pack_pair(lo, hi):
    lo16 = jax.lax.bitcast_convert_type(lo.astype(jnp.bfloat16),
                                        jnp.uint16).astype(jnp.uint32)
    hi16 = jax.lax.bitcast_convert_type(hi.astype(jnp.bfloat16),
                                        jnp.uint16).astype(jnp.uint32)
    return ((hi16 << 16) | lo16).astype(jnp.int32)


def _pack_256(o):
    return jnp.concatenate(
        [_pack_pair(o[:, 0:64], o[:, 64:128]),
         _pack_pair(o[:, 128:192], o[:, 192:256])], axis=1)


def _pre_body(x_ref, w_ref, b_ref, o_ref, hp_ref):
    o = (jnp.dot(x_ref[...], w_ref[0],
                 preferred_element_type=jnp.float32) + b_ref[0])
    o_ref[0] = o[:, 0:128]
    o_ref[1] = o[:, 128:256]
    hp_ref[0] = _pack_256(o)


@jax.jit
def _pre(x, w2, b2):
    return pl.pallas_call(
        _pre_body,
        grid=(2,),
        in_specs=[
            pl.BlockSpec((N, 128), lambda c: (0, 0)),
            pl.BlockSpec((1, 128, 2 * DC), lambda c: (c, 0, 0)),
            pl.BlockSpec((1, 1, 2 * DC), lambda c: (c, 0, 0)),
        ],
        out_specs=[
            pl.BlockSpec((2, N, DC), lambda c: (c, 0, 0)),
            pl.BlockSpec((1, N, DC), lambda c: (c, 0, 0)),
        ],
        out_shape=[
            jax.ShapeDtypeStruct((NSL, N, DC), jnp.float32),
            jax.ShapeDtypeStruct((2, N, DC), jnp.int32),
        ],
    )(x, w2, b2)


_BE = 8000


def _edge_body(ea_ref, w_ref, b_ref, o_ref):
    o = (jnp.dot(ea_ref[...], w_ref[0],
                 preferred_element_type=jnp.float32) + b_ref[0])
    o_ref[0] = _pack_256(o)


@jax.jit
def _edge(ea, w2, b2):
    return pl.pallas_call(
        _edge_body,
        grid=(2, E // _BE),
        in_specs=[
            pl.BlockSpec((_BE, 16), lambda c, j: (j, 0)),
            pl.BlockSpec((1, 16, 2 * DC), lambda c, j: (c, 0, 0)),
            pl.BlockSpec((1, 1, 2 * DC), lambda c, j: (c, 0, 0)),
        ],
        out_specs=pl.BlockSpec((1, _BE, DC), lambda c, j: (c, j, 0)),
        out_shape=jax.ShapeDtypeStruct((2, E, DC), jnp.int32),
    )(ea, w2, b2)


_BN = 1000


def _mlp_body(eps_ref, h_ref, a_ref, w1_ref, b1_ref, w2_ref, b2_ref, o_ref,
              hp_ref):
    ep = eps_ref[0, 0]
    relu_flag = eps_ref[0, 1]
    z1 = jnp.dot(ep * h_ref[0] + a_ref[0], w1_ref[0],
                 preferred_element_type=jnp.float32)
    for c in range(1, NSL):
        z1 += jnp.dot(ep * h_ref[c] + a_ref[c], w1_ref[c],
                      preferred_element_type=jnp.float32)
    z1 = jnp.maximum(z1 + b1_ref[...], 0.0)
    os = []
    for c in range(NSL):
        o = (jnp.dot(z1, w2_ref[c], preferred_element_type=jnp.float32)
             + b2_ref[c:c + 1, :])
        o = jnp.where(relu_flag > 0.5, jnp.maximum(o, 0.0), o)
        o_ref[c] = o
        os.append(o)
    for c2 in range(2):
        hp_ref[c2] = _pack_256(jnp.concatenate(
            [os[2 * c2], os[2 * c2 + 1]], axis=1))


@jax.jit
def _mlp(scal, h4, agg4, w1r, b1f, w2r, b2f):
    return pl.pallas_call(
        _mlp_body,
        grid=(N // _BN,),
        in_specs=[
            pl.BlockSpec(memory_space=pltpu.SMEM),
            pl.BlockSpec((NSL, _BN, DC), lambda i: (0, i, 0)),
            pl.BlockSpec((NSL, _BN, DC), lambda i: (0, i, 0)),
            pl.BlockSpec((NSL, DC, 2 * D), lambda i: (0, 0, 0)),
            pl.BlockSpec((1, 2 * D), lambda i: (0, 0)),
            pl.BlockSpec((NSL, 2 * D, DC), lambda i: (0, 0, 0)),
            pl.BlockSpec((NSL, DC), lambda i: (0, 0)),
        ],
        out_specs=[
            pl.BlockSpec((NSL, _BN, DC), lambda i: (0, i, 0)),
            pl.BlockSpec((2, _BN, DC), lambda i: (0, i, 0)),
        ],
        out_shape=[
            jax.ShapeDtypeStruct((NSL, N, DC), jnp.float32),
            jax.ShapeDtypeStruct((2, N, DC), jnp.int32),
        ],
    )(scal, h4, agg4, w1r, b1f, w2r, b2f)


def _pool_body(b_ref, h_ref, wo_ref, bo_ref, o_ref, sums, cnt):
    i = pl.program_id(0)

    @pl.when(i == 0)
    def _init():
        sums[...] = jnp.zeros_like(sums)
        cnt[...] = jnp.zeros_like(cnt)

    pt = (lax.broadcasted_iota(jnp.int32, (G, _BN), 0)
          == b_ref[0]).astype(jnp.float32)
    cnt[...] += jnp.dot(pt, jnp.ones((_BN, 128), jnp.float32),
                        preferred_element_type=jnp.float32)
    for c in range(NSL):
        sums[c] += jnp.dot(pt, h_ref[c], preferred_element_type=jnp.float32)

    @pl.when(i == N // _BN - 1)
    def _fin():
        inv = 1.0 / jnp.maximum(cnt[...], 1.0)
        o = bo_ref[...] + jnp.dot(sums[0] * inv, wo_ref[0],
                                  preferred_element_type=jnp.float32)
        for c in range(1, NSL):
            o += jnp.dot(sums[c] * inv, wo_ref[c],
                         preferred_element_type=jnp.float32)
        o_ref[...] = o


@jax.jit
def _pool(batch2, h4, wo4, bo2):
    return pl.pallas_call(
        _pool_body,
        grid=(N // _BN,),
        in_specs=[
            pl.BlockSpec((1, 1, _BN), lambda i: (i, 0, 0)),
            pl.BlockSpec((NSL, _BN, DC), lambda i: (0, i, 0)),
            pl.BlockSpec((NSL, DC, NL), lambda i: (0, 0, 0)),
            pl.BlockSpec((1, NL), lambda i: (0, 0)),
        ],
        out_specs=pl.BlockSpec((G, NL), lambda i: (0, 0)),
        out_shape=jax.ShapeDtypeStruct((G, NL), jnp.float32),
        scratch_shapes=[
            pltpu.VMEM((NSL, G, DC), jnp.float32),
            pltpu.VMEM((G, DC), jnp.float32),
        ],
    )(batch2, h4, wo4, bo2)


def kernel(x, edge_index, edge_attr, batch, W_pre, b_pre, W1, b1, bn1_g,
           bn1_b, W2, b2, We, be, bn_g, bn_b, eps_gin, W_out, b_out):
    f32 = jnp.float32
    src = edge_index[0]
    dst = edge_index[1]
    src8 = (2 * src.reshape(1, 1, NS, CH, K)
            + jnp.arange(2, dtype=jnp.int32).reshape(1, 2, 1, 1, 1)
            + (jnp.arange(2, dtype=jnp.int32) * 2 * N).reshape(2, 1, 1, 1, 1))
    dst3 = dst.reshape(NS, CH, K)
    batch2 = batch.reshape(N // _BN, 1, _BN)

    wpre2 = W_pre.reshape(128, 2, 2 * DC).transpose(1, 0, 2)
    bpre2 = b_pre.reshape(2, 1, 2 * DC)
    w1f = W1 * bn1_g[:, None, :]
    b1f = b1 * bn1_g + bn1_b
    w2f = W2 * bn_g[:, None, :]
    b2f = b2 * bn_g + bn_b
    w1r = w1f.reshape(L, NSL, DC, 2 * D)
    w2r = w2f.reshape(L, 2 * D, NSL, DC).transpose(0, 2, 1, 3)
    we2 = We.reshape(L, 16, 2, 2 * DC).transpose(0, 2, 1, 3)
    be2 = be.reshape(L, 2, 1, 2 * DC)
    wo4 = W_out.reshape(NSL, DC, NL)
    bo2 = b_out.reshape(1, NL)
    b1f2 = b1f.reshape(L, 1, 2 * D)
    b2f2 = b2f.reshape(L, NSL, DC)
    relu_flags = (jnp.arange(L) < L - 1).astype(f32)
    scal = jnp.stack([1.0 + eps_gin.astype(f32), relu_flags], axis=1)

    h4, hp = _pre(x, wpre2, bpre2)

    def layer(l, carry):
        h4, hp = carry
        idx = functools.partial(lax.dynamic_index_in_dim, index=l, axis=0,
                                keepdims=False)
        ep = _edge(edge_attr, idx(we2), idx(be2))
        agg4 = _sc_agg(hp.reshape(2 * N * 2, W), ep, src8, dst3)
        h4n, hpn = _mlp(idx(scal).reshape(1, 2), h4, agg4, idx(w1r),
                        idx(b1f2), idx(w2r), idx(b2f2))
        return (h4n, hpn)

    h4, hp = lax.fori_loop(0, L, layer, (h4, hp))
    return _pool(batch2, h4, wo4, bo2)
